# baseline (speedup 1.0000x reference)
import functools

import jax
import jax.numpy as jnp
from jax import lax
from jax.experimental import pallas as pl
from jax.experimental.pallas import tpu as pltpu
from jax.experimental.pallas import tpu_sc as plsc

N_NODES = 10000
D = 128
NS = 16
CHUNK = 128
NPH = 2
PCPW = 80
CPW = NPH * PCPW
EPW = CHUNK * CPW
E_PAD = NS * EPW
ACC_ROWS = 10112
ROWS_PER_TILE = ACC_ROWS // NS
PAD_DST = N_NODES

BM = 1000


def _seg_sum_sc(h, packed4):
    mesh = plsc.VectorSubcoreMesh(core_axis_name="c", subcore_axis_name="s")

    @functools.partial(
        pl.kernel,
        out_type=jax.ShapeDtypeStruct((ACC_ROWS, D), jnp.float32),
        mesh=mesh,
        scratch_types=[
            pltpu.VMEM((PCPW, CHUNK), jnp.int32),
            pltpu.VMEM((CHUNK,), jnp.int32),
            pltpu.VMEM((CHUNK,), jnp.int32),
            pltpu.VMEM((CHUNK,), jnp.int32),
            pltpu.VMEM((CHUNK,), jnp.int32),
            pltpu.VMEM((CHUNK, D), jnp.float32),
            pltpu.VMEM((CHUNK, D), jnp.float32),
            pltpu.VMEM_SHARED((ACC_ROWS, D), jnp.float32),
            pltpu.SemaphoreType.DMA,
            pltpu.SemaphoreType.DMA,
        ],
    )
    def k(h_hbm, packed_hbm, out_hbm, packed_v, sbuf0, sbuf1, dbuf0, dbuf1,
          msg0, msg1, acc, sem0, sem1):
        cid = lax.axis_index("c")
        sid = lax.axis_index("s")
        my_packed = packed_hbm.at[sid]

        def unpack(c, sbuf, dbuf):
            for j in range(CHUNK // 16):
                v = packed_v[c, pl.ds(j * 16, 16)]
                sbuf[pl.ds(j * 16, 16)] = lax.bitwise_and(v, 0xFFFF)
                dbuf[pl.ds(j * 16, 16)] = lax.shift_right_logical(v, 16)

        @pl.when(cid == 1)
        def _shadow():
            pltpu.sync_copy(my_packed.at[0], packed_v)
            unpack(0, sbuf0, dbuf0)
            pltpu.async_copy(h_hbm.at[sbuf0], msg0, sem0)
            unpack(1, sbuf1, dbuf1)
            pltpu.async_copy(h_hbm.at[sbuf1], msg1, sem1)

            def sbody(i, carry):
                pltpu.make_async_copy(h_hbm.at[sbuf0], msg0, sem0).wait()
                pltpu.async_copy(h_hbm.at[sbuf0], msg0, sem0)
                pltpu.make_async_copy(h_hbm.at[sbuf1], msg1, sem1).wait()
                pltpu.async_copy(h_hbm.at[sbuf1], msg1, sem1)
                return carry

            lax.fori_loop(0, CPW // 2 - 1, sbody, 0)
            pltpu.make_async_copy(h_hbm.at[sbuf0], msg0, sem0).wait()
            pltpu.make_async_copy(h_hbm.at[sbuf1], msg1, sem1).wait()

        @pl.when(cid == 0)
        def _work():
            zero = jnp.zeros((16,), jnp.float32)

            def zrow(i, carry):
                for j in range(D // 16):
                    msg0[i, pl.ds(j * 16, 16)] = zero
                return carry

            lax.fori_loop(0, CHUNK, zrow, 0)
            base = sid * ROWS_PER_TILE
            parts = []
            off = 0
            while off < ROWS_PER_TILE:
                ln = min(CHUNK, ROWS_PER_TILE - off)
                parts.append((off, ln))
                off += ln
            for off, ln in parts:
                pltpu.async_copy(msg0.at[pl.ds(0, ln)],
                                 acc.at[pl.ds(base + off, ln)], sem0)
            for off, ln in parts:
                pltpu.make_async_copy(msg0.at[pl.ds(0, ln)],
                                      acc.at[pl.ds(base + off, ln)],
                                      sem0).wait()
            plsc.subcore_barrier()

            def body(i, carry):
                c = i * 2
                pltpu.make_async_copy(h_hbm.at[sbuf0], msg0, sem0).wait()
                pltpu.sync_copy(msg0, acc.at[dbuf0], add=True)
                unpack(jnp.minimum(c + 2, PCPW - 1), sbuf0, dbuf0)
                pltpu.async_copy(h_hbm.at[sbuf0], msg0, sem0)
                pltpu.make_async_copy(h_hbm.at[sbuf1], msg1, sem1).wait()
                pltpu.sync_copy(msg1, acc.at[dbuf1], add=True)
                unpack(jnp.minimum(c + 3, PCPW - 1), sbuf1, dbuf1)
                pltpu.async_copy(h_hbm.at[sbuf1], msg1, sem1)
                return carry

            for ph in range(NPH):
                pltpu.sync_copy(my_packed.at[ph], packed_v)
                unpack(0, sbuf0, dbuf0)
                pltpu.async_copy(h_hbm.at[sbuf0], msg0, sem0)
                unpack(1, sbuf1, dbuf1)
                pltpu.async_copy(h_hbm.at[sbuf1], msg1, sem1)
                lax.fori_loop(0, PCPW // 2, body, 0)
                pltpu.make_async_copy(h_hbm.at[sbuf0], msg0, sem0).wait()
                pltpu.make_async_copy(h_hbm.at[sbuf1], msg1, sem1).wait()

            plsc.subcore_barrier()

            msgs = (msg0, msg1)
            sems = (sem0, sem1)
            for q, (off, ln) in enumerate(parts):
                m = q % 2
                if q >= 2:
                    poff, pln = parts[q - 2]
                    pltpu.make_async_copy(
                        msgs[m].at[pl.ds(0, pln)],
                        out_hbm.at[pl.ds(base + poff, pln)],
                        sems[m]).wait()
                pltpu.sync_copy(acc.at[pl.ds(base + off, ln)],
                                msgs[m].at[pl.ds(0, ln)])
                pltpu.async_copy(msgs[m].at[pl.ds(0, ln)],
                                 out_hbm.at[pl.ds(base + off, ln)], sems[m])
            nparts = len(parts)
            for q in range(max(0, nparts - 2), nparts):
                off, ln = parts[q]
                pltpu.make_async_copy(msgs[q % 2].at[pl.ds(0, ln)],
                                      out_hbm.at[pl.ds(base + off, ln)],
                                      sems[q % 2]).wait()

    return k(h, packed4)


def _mm(x, W):
    M = x.shape[0]

    def kfn(x_ref, w_ref, o_ref):
        o_ref[...] = jnp.dot(x_ref[...], w_ref[...],
                             preferred_element_type=jnp.float32)

    return pl.pallas_call(
        kfn,
        grid=(M // BM,),
        in_specs=[pl.BlockSpec((BM, D), lambda i: (i, 0)),
                  pl.BlockSpec((D, D), lambda i: (0, 0))],
        out_specs=pl.BlockSpec((BM, D), lambda i: (i, 0)),
        out_shape=jax.ShapeDtypeStruct((M, D), jnp.float32),
    )(x, W)


def _comb_mm(acc, b2d, W):
    def kfn(a_ref, b_ref, w_ref, o_ref):
        h = jnp.maximum(a_ref[...] + b_ref[...], 0.0)
        o_ref[...] = jnp.dot(h, w_ref[...],
                             preferred_element_type=jnp.float32)

    return pl.pallas_call(
        kfn,
        grid=(N_NODES // BM,),
        in_specs=[pl.BlockSpec((BM, D), lambda i: (i, 0)),
                  pl.BlockSpec((1, D), lambda i: (0, 0)),
                  pl.BlockSpec((D, D), lambda i: (0, 0))],
        out_specs=pl.BlockSpec((BM, D), lambda i: (i, 0)),
        out_shape=jax.ShapeDtypeStruct((N_NODES, D), jnp.float32),
    )(acc, b2d, W)


def _comb(acc, b2d):
    def kfn(a_ref, b_ref, o_ref):
        o_ref[...] = jnp.maximum(a_ref[...] + b_ref[...], 0.0)

    return pl.pallas_call(
        kfn,
        grid=(N_NODES // BM,),
        in_specs=[pl.BlockSpec((BM, D), lambda i: (i, 0)),
                  pl.BlockSpec((1, D), lambda i: (0, 0))],
        out_specs=pl.BlockSpec((BM, D), lambda i: (i, 0)),
        out_shape=jax.ShapeDtypeStruct((N_NODES, D), jnp.float32),
    )(acc, b2d)


def kernel(x, edge_index, W1, b1, W2, b2):
    src = edge_index[0].astype(jnp.int32)
    dst = edge_index[1].astype(jnp.int32)
    n_edges = src.shape[0]
    pad = E_PAD - n_edges
    packed = jnp.bitwise_or(jnp.left_shift(dst, 16), src)
    packed = jnp.concatenate(
        [packed, jnp.full((pad,), PAD_DST << 16, jnp.int32)])
    packed4 = packed.reshape(NS, NPH, PCPW, CHUNK)
    b1r = b1.reshape(1, D)
    b2r = b2.reshape(1, D)

    h1 = _mm(x, W1)
    acc1 = _seg_sum_sc(h1, packed4)
    h2 = _comb_mm(acc1, b1r, W2)
    acc2 = _seg_sum_sc(h2, packed4)
    return _comb(acc2, b2r)


# speedup vs baseline: 1.2271x; 1.2271x over previous
import functools

import jax
import jax.numpy as jnp
from jax import lax
from jax.experimental import pallas as pl
from jax.experimental.pallas import tpu as pltpu
from jax.experimental.pallas import tpu_sc as plsc

N_NODES = 10000
D = 128
NC = 2
NS = 16
CHUNK = 128
CPW0 = 120
CPW1 = 40
E0 = NS * CPW0 * CHUNK
E1 = NS * CPW1 * CHUNK
E_PAD = E0 + E1
ACC_ROWS = 10112
ROWS_PER_TILE = ACC_ROWS // NS
PAD_DST = N_NODES

BM = 1000


def _seg_sum_sc(h, packed4):
    mesh = plsc.VectorSubcoreMesh(core_axis_name="c", subcore_axis_name="s")

    @functools.partial(
        pl.kernel,
        out_type=jax.ShapeDtypeStruct((NC, ACC_ROWS, D), jnp.float32),
        mesh=mesh,
        scratch_types=[
            pltpu.VMEM((CPW0, CHUNK), jnp.int32),
            pltpu.VMEM((CHUNK,), jnp.int32),
            pltpu.VMEM((CHUNK,), jnp.int32),
            pltpu.VMEM((CHUNK,), jnp.int32),
            pltpu.VMEM((CHUNK,), jnp.int32),
            pltpu.VMEM((CHUNK, D), jnp.float32),
            pltpu.VMEM((CHUNK, D), jnp.float32),
            pltpu.VMEM_SHARED((ACC_ROWS, D), jnp.float32),
            pltpu.SemaphoreType.DMA,
            pltpu.SemaphoreType.DMA,
        ],
    )
    def k(h_hbm, packed_hbm, out_hbm, packed_v, sbuf0, sbuf1, dbuf0, dbuf1,
          msg0, msg1, acc, sem0, sem1):
        cid = lax.axis_index("c")
        sid = lax.axis_index("s")
        nchunks = jnp.where(cid == 0, CPW0, CPW1)
        last = nchunks - 1

        pltpu.sync_copy(packed_hbm.at[cid].at[sid], packed_v)

        zero = jnp.zeros((16,), jnp.float32)

        def zrow(i, carry):
            for j in range(D // 16):
                msg0[i, pl.ds(j * 16, 16)] = zero
            return carry

        lax.fori_loop(0, CHUNK, zrow, 0)
        base = sid * ROWS_PER_TILE
        parts = []
        off = 0
        while off < ROWS_PER_TILE:
            ln = min(CHUNK, ROWS_PER_TILE - off)
            parts.append((off, ln))
            off += ln
        for off, ln in parts:
            pltpu.async_copy(msg0.at[pl.ds(0, ln)],
                             acc.at[pl.ds(base + off, ln)], sem0)
        for off, ln in parts:
            pltpu.make_async_copy(msg0.at[pl.ds(0, ln)],
                                  acc.at[pl.ds(base + off, ln)],
                                  sem0).wait()
        plsc.subcore_barrier()

        def unpack(c, sbuf, dbuf):
            for j in range(CHUNK // 16):
                v = packed_v[c, pl.ds(j * 16, 16)]
                sbuf[pl.ds(j * 16, 16)] = lax.bitwise_and(v, 0xFFFF)
                dbuf[pl.ds(j * 16, 16)] = lax.shift_right_logical(v, 16)

        unpack(0, sbuf0, dbuf0)
        pltpu.async_copy(h_hbm.at[sbuf0], msg0, sem0)
        unpack(jnp.minimum(1, last), sbuf1, dbuf1)
        pltpu.async_copy(h_hbm.at[sbuf1], msg1, sem1)

        def body(i, carry):
            c = i * 2
            pltpu.make_async_copy(h_hbm.at[sbuf0], msg0, sem0).wait()
            pltpu.sync_copy(msg0, acc.at[dbuf0], add=True)
            unpack(jnp.minimum(c + 2, last), sbuf0, dbuf0)
            pltpu.async_copy(h_hbm.at[sbuf0], msg0, sem0)
            pltpu.make_async_copy(h_hbm.at[sbuf1], msg1, sem1).wait()
            pltpu.sync_copy(msg1, acc.at[dbuf1], add=True)
            unpack(jnp.minimum(c + 3, last), sbuf1, dbuf1)
            pltpu.async_copy(h_hbm.at[sbuf1], msg1, sem1)
            return carry

        lax.fori_loop(0, nchunks // 2, body, 0)
        pltpu.make_async_copy(h_hbm.at[sbuf0], msg0, sem0).wait()
        pltpu.make_async_copy(h_hbm.at[sbuf1], msg1, sem1).wait()
        plsc.subcore_barrier()

        my_out = out_hbm.at[cid]
        msgs = (msg0, msg1)
        sems = (sem0, sem1)
        for q, (off, ln) in enumerate(parts):
            m = q % 2
            if q >= 2:
                poff, pln = parts[q - 2]
                pltpu.make_async_copy(
                    msgs[m].at[pl.ds(0, pln)],
                    my_out.at[pl.ds(base + poff, pln)], sems[m]).wait()
            pltpu.sync_copy(acc.at[pl.ds(base + off, ln)],
                            msgs[m].at[pl.ds(0, ln)])
            pltpu.async_copy(msgs[m].at[pl.ds(0, ln)],
                             my_out.at[pl.ds(base + off, ln)], sems[m])
        nparts = len(parts)
        for q in range(max(0, nparts - 2), nparts):
            off, ln = parts[q]
            pltpu.make_async_copy(msgs[q % 2].at[pl.ds(0, ln)],
                                  my_out.at[pl.ds(base + off, ln)],
                                  sems[q % 2]).wait()

    return k(h, packed4)


def _mm(x, W):
    M = x.shape[0]

    def kfn(x_ref, w_ref, o_ref):
        o_ref[...] = jnp.dot(x_ref[...], w_ref[...],
                             preferred_element_type=jnp.float32)

    return pl.pallas_call(
        kfn,
        grid=(M // BM,),
        in_specs=[pl.BlockSpec((BM, D), lambda i: (i, 0)),
                  pl.BlockSpec((D, D), lambda i: (0, 0))],
        out_specs=pl.BlockSpec((BM, D), lambda i: (i, 0)),
        out_shape=jax.ShapeDtypeStruct((M, D), jnp.float32),
    )(x, W)


def _comb_mm(acc, b2d, W):
    def kfn(a0_ref, a1_ref, b_ref, w_ref, o_ref):
        h = jnp.maximum(a0_ref[0] + a1_ref[0] + b_ref[...], 0.0)
        o_ref[...] = jnp.dot(h, w_ref[...],
                             preferred_element_type=jnp.float32)

    return pl.pallas_call(
        kfn,
        grid=(N_NODES // BM,),
        in_specs=[pl.BlockSpec((1, BM, D), lambda i: (0, i, 0)),
                  pl.BlockSpec((1, BM, D), lambda i: (1, i, 0)),
                  pl.BlockSpec((1, D), lambda i: (0, 0)),
                  pl.BlockSpec((D, D), lambda i: (0, 0))],
        out_specs=pl.BlockSpec((BM, D), lambda i: (i, 0)),
        out_shape=jax.ShapeDtypeStruct((N_NODES, D), jnp.float32),
    )(acc, acc, b2d, W)


def _comb(acc, b2d):
    def kfn(a0_ref, a1_ref, b_ref, o_ref):
        o_ref[...] = jnp.maximum(a0_ref[0] + a1_ref[0] + b_ref[...], 0.0)

    return pl.pallas_call(
        kfn,
        grid=(N_NODES // BM,),
        in_specs=[pl.BlockSpec((1, BM, D), lambda i: (0, i, 0)),
                  pl.BlockSpec((1, BM, D), lambda i: (1, i, 0)),
                  pl.BlockSpec((1, D), lambda i: (0, 0))],
        out_specs=pl.BlockSpec((BM, D), lambda i: (i, 0)),
        out_shape=jax.ShapeDtypeStruct((N_NODES, D), jnp.float32),
    )(acc, acc, b2d)


def kernel(x, edge_index, W1, b1, W2, b2):
    src = edge_index[0].astype(jnp.int32)
    dst = edge_index[1].astype(jnp.int32)
    n_edges = src.shape[0]
    pad = E_PAD - n_edges
    packed = jnp.bitwise_or(jnp.left_shift(dst, 16), src)
    packed = jnp.concatenate(
        [packed, jnp.full((pad,), PAD_DST << 16, jnp.int32)])
    p0 = packed[:E0].reshape(NS, CPW0, CHUNK)
    p1 = packed[E0:].reshape(NS, CPW1, CHUNK)
    p1 = jnp.pad(p1, ((0, 0), (0, CPW0 - CPW1), (0, 0)),
                 constant_values=PAD_DST << 16)
    packed4 = jnp.stack([p0, p1])
    b1r = b1.reshape(1, D)
    b2r = b2.reshape(1, D)

    h1 = _mm(x, W1)
    acc1 = _seg_sum_sc(h1, packed4)
    h2 = _comb_mm(acc1, b1r, W2)
    acc2 = _seg_sum_sc(h2, packed4)
    return _comb(acc2, b2r)
